# Initial kernel scaffold; baseline (speedup 1.0000x reference)
import jax
import jax.numpy as jnp
from jax.experimental import pallas as pl


def kernel(x, pos, edge_index, ln1_w, ln1_b, ln2_w, ln2_b, gn1_w, gn1_b, gn2_w, gn2_b, gn3_w, gn3_b, gat_w, gat_asrc, gat_adst, gat_b, gcn1_w, gcn1_b, gcn2_w, gcn2_b, out_w, out_b):
    raise NotImplementedError("write your pallas kernel here")


# speedup vs baseline: 1.4388x; 1.4388x over previous
import jax
import jax.numpy as jnp
from jax.experimental import pallas as pl


def _copy_body(x_ref, o_ref):
    o_ref[...] = x_ref[...]


def kernel(x, pos, edge_index, ln1_w, ln1_b, ln2_w, ln2_b, gn1_w, gn1_b, gn2_w, gn2_b, gn3_w, gn3_b, gat_w, gat_asrc, gat_adst, gat_b, gcn1_w, gcn1_b, gcn2_w, gcn2_b, out_w, out_b):
    n, d = x.shape
    loops = jnp.arange(n, dtype=edge_index.dtype)
    ei = jnp.concatenate([edge_index, jnp.stack([loops, loops])], axis=1)
    src, dst = ei[0], ei[1]

    u = x @ ln1_w[:d] + pos @ ln1_w[d:] + ln1_b
    v = pos @ ln1_w[d:]
    r = jax.nn.relu(u[src] - v[dst])
    h2 = r @ ln2_w
    agg = jax.ops.segment_max(h2, dst, num_segments=n) + ln2_b

    g = jax.nn.relu(agg @ gn1_w + gn1_b)
    g = jax.nn.relu(g @ gn2_w + gn2_b)
    h = g @ gn3_w + gn3_b

    xw = h @ gat_w
    a_s = (xw * gat_asrc).sum(-1)
    a_d = (xw * gat_adst).sum(-1)
    m = jnp.max(a_s) + jnp.max(a_d)
    a = jax.nn.leaky_relu(a_s[src] + a_d[dst], 0.2)
    ae = jnp.exp(a - m)
    denom = jax.ops.segment_sum(ae, dst, num_segments=n)
    num = jax.ops.segment_sum(ae[:, None] * xw[src], dst, num_segments=n)
    h = jax.nn.relu(num / denom[:, None] + gat_b)

    deg = jax.ops.segment_sum(jnp.ones((src.shape[0],), h.dtype), dst, num_segments=n)
    dinv = jnp.where(deg > 0, deg ** -0.5, 0.0)

    def gcn(hh, W, b):
        q = jax.ops.segment_sum((dinv[:, None] * hh)[src], dst, num_segments=n)
        return (dinv[:, None] * q) @ W + b

    h = jax.nn.relu(gcn(h, gcn1_w, gcn1_b))
    h = jax.nn.relu(gcn(h, gcn2_w, gcn2_b))
    out = gcn(h, out_w, out_b)

    return pl.pallas_call(
        _copy_body,
        out_shape=jax.ShapeDtypeStruct(out.shape, out.dtype),
    )(out)


# speedup vs baseline: 2.0785x; 1.4446x over previous
import jax
import jax.numpy as jnp
from jax import lax
from jax.experimental import pallas as pl
from jax.experimental.pallas import tpu as pltpu
from jax.experimental.pallas import tpu_sc as plsc

_N = 10000
_NP = 10240
_NPT = _NP // 32
_ECH = 128
_EPAD = 331776
_EPT = _EPAD // 32
_NCH = _EPT // _ECH
_PADN = 10008


def _apass_body(p_hbm, src_hbm, dst_hbm, zero_hbm, out_hbm,
                sidx, didx, rows, sem, acc):
    c = lax.axis_index("c")
    s = lax.axis_index("s")
    nps = _NP // 16
    slab = s * nps
    pltpu.sync_copy(zero_hbm.at[pl.ds(slab, nps)], acc.at[pl.ds(slab, nps)])
    plsc.subcore_barrier()
    base0 = (c * 16 + s) * _EPT

    def body(k, carry):
        base = base0 + k * _ECH
        pltpu.sync_copy(src_hbm.at[pl.ds(base, _ECH)], sidx)
        pltpu.sync_copy(dst_hbm.at[pl.ds(base, _ECH)], didx)
        pltpu.async_copy(p_hbm.at[sidx], rows, sem).wait()
        pltpu.sync_copy(rows, acc.at[didx], add=True)
        return carry

    lax.fori_loop(0, _NCH, body, 0)
    plsc.subcore_barrier()
    pltpu.sync_copy(acc.at[pl.ds(slab, nps)], out_hbm.at[c, pl.ds(slab, nps)])


_apass = pl.kernel(
    _apass_body,
    out_type=jax.ShapeDtypeStruct((2, _NP, 128), jnp.float32),
    mesh=plsc.VectorSubcoreMesh(core_axis_name="c", subcore_axis_name="s"),
    scratch_types=[
        pltpu.VMEM((_ECH,), jnp.int32),
        pltpu.VMEM((_ECH,), jnp.int32),
        pltpu.VMEM((_ECH, 128), jnp.float32),
        pltpu.SemaphoreType.DMA,
        pltpu.VMEM_SHARED((_NP, 128), jnp.float32),
    ],
)


def kernel(x, pos, edge_index, ln1_w, ln1_b, ln2_w, ln2_b, gn1_w, gn1_b, gn2_w, gn2_b, gn3_w, gn3_b, gat_w, gat_asrc, gat_adst, gat_b, gcn1_w, gcn1_b, gcn2_w, gcn2_b, out_w, out_b):
    n, d = x.shape
    loops = jnp.arange(n, dtype=edge_index.dtype)
    ei = jnp.concatenate([edge_index, jnp.stack([loops, loops])], axis=1)
    src, dst = ei[0], ei[1]
    pad = jnp.full((_EPAD - src.shape[0],), _PADN, dtype=jnp.int32)
    srcp = jnp.concatenate([src.astype(jnp.int32), pad])
    dstp = jnp.concatenate([dst.astype(jnp.int32), pad])
    zeros_np = jnp.zeros((_NP, 128), jnp.float32)

    u = x @ ln1_w[:d] + pos @ ln1_w[d:] + ln1_b
    v = pos @ ln1_w[d:]
    r = jax.nn.relu(u[src] - v[dst])
    h2 = r @ ln2_w
    agg = jax.ops.segment_max(h2, dst, num_segments=n) + ln2_b

    g = jax.nn.relu(agg @ gn1_w + gn1_b)
    g = jax.nn.relu(g @ gn2_w + gn2_b)
    h = g @ gn3_w + gn3_b

    xw = h @ gat_w
    a_s = (xw * gat_asrc).sum(-1)
    a_d = (xw * gat_adst).sum(-1)
    t = jnp.max(a_s) + jnp.max(a_d)
    m = jnp.where(t > 0, t, 0.2 * t)
    a = jax.nn.leaky_relu(a_s[src] + a_d[dst], 0.2)
    ae = jnp.exp(a - m)
    denom = jax.ops.segment_sum(ae, dst, num_segments=n)
    num = jax.ops.segment_sum(ae[:, None] * xw[src], dst, num_segments=n)
    h = jax.nn.relu(num / denom[:, None] + gat_b)

    deg = jax.ops.segment_sum(jnp.ones((src.shape[0],), h.dtype), dst, num_segments=n)
    dinv = jnp.where(deg > 0, deg ** -0.5, 0.0)

    def gcn(hh, W, b):
        p = jnp.pad(dinv[:, None] * hh, ((0, _NP - n), (0, 0)))
        qh = _apass(p, srcp, dstp, zeros_np)
        q = (qh[0] + qh[1])[:n]
        return (dinv[:, None] * q) @ W + b

    h = jax.nn.relu(gcn(h, gcn1_w, gcn1_b))
    h = jax.nn.relu(gcn(h, gcn2_w, gcn2_b))
    return gcn(h, out_w, out_b)


# speedup vs baseline: 6.1530x; 2.9603x over previous
import jax
import jax.numpy as jnp
from jax import lax
from jax.experimental import pallas as pl
from jax.experimental.pallas import tpu as pltpu
from jax.experimental.pallas import tpu_sc as plsc

_N = 10000
_NP = 10240
_ECH = 128
_NCH = 82
_EPT = _NCH * _ECH
_EPAD = 32 * _EPT
_PADN = 10008
_MESH = plsc.VectorSubcoreMesh(core_axis_name="c", subcore_axis_name="s")


def _prelude(z_hbm, acc, s, width):
    nps = _NP // 16
    slab = s * nps
    pltpu.sync_copy(z_hbm.at[pl.ds(slab, nps)], acc.at[pl.ds(slab, nps)])
    plsc.subcore_barrier()
    return slab, nps


def _epilogue(acc, out_hbm, c, slab, nps):
    plsc.subcore_barrier()
    pltpu.sync_copy(acc.at[pl.ds(slab, nps)], out_hbm.at[c, pl.ds(slab, nps)])


def _apass_body(p_hbm, src_hbm, dst_hbm, zero_hbm, out_hbm,
                sidx0, sidx1, didx0, didx1, rows0, rows1,
                ss0, ss1, sd0, sd1, gr0, gr1, acc):
    c = lax.axis_index("c")
    s = lax.axis_index("s")
    slab, nps = _prelude(zero_hbm, acc, s, 128)
    base0 = (c * 16 + s) * _EPT
    sidx = (sidx0, sidx1)
    didx = (didx0, didx1)
    rows = (rows0, rows1)
    ssem = (ss0, ss1)
    dsem = (sd0, sd1)
    rsem = (gr0, gr1)

    def idx_load(k, b):
        pltpu.async_copy(src_hbm.at[pl.ds(base0 + k * _ECH, _ECH)], sidx[b], ssem[b])
        pltpu.async_copy(dst_hbm.at[pl.ds(base0 + k * _ECH, _ECH)], didx[b], dsem[b])

    def idx_wait(b):
        pltpu.make_async_copy(src_hbm.at[pl.ds(0, _ECH)], sidx[b], ssem[b]).wait()
        pltpu.make_async_copy(dst_hbm.at[pl.ds(0, _ECH)], didx[b], dsem[b]).wait()

    def gath(b):
        pltpu.async_copy(p_hbm.at[sidx[b]], rows[b], rsem[b])

    def gath_wait(b):
        pltpu.make_async_copy(p_hbm.at[sidx[b]], rows[b], rsem[b]).wait()

    idx_load(0, 0)
    idx_wait(0)
    gath(0)
    idx_load(1, 1)

    def pair(kk, carry):
        for b in (0, 1):
            k = 2 * kk + b
            nb = 1 - b
            gath_wait(b)

            @pl.when(k + 1 < _NCH)
            def _():
                idx_wait(nb)
                gath(nb)

            pltpu.sync_copy(rows[b], acc.at[didx[b]], add=True)

            @pl.when(k + 2 < _NCH)
            def _():
                idx_load(k + 2, b)
        return carry

    lax.fori_loop(0, _NCH // 2, pair, 0)
    _epilogue(acc, out_hbm, c, slab, nps)


_apass = pl.kernel(
    _apass_body,
    out_type=jax.ShapeDtypeStruct((2, _NP, 128), jnp.float32),
    mesh=_MESH,
    scratch_types=[
        pltpu.VMEM((_ECH,), jnp.int32), pltpu.VMEM((_ECH,), jnp.int32),
        pltpu.VMEM((_ECH,), jnp.int32), pltpu.VMEM((_ECH,), jnp.int32),
        pltpu.VMEM((_ECH, 128), jnp.float32), pltpu.VMEM((_ECH, 128), jnp.float32),
        pltpu.SemaphoreType.DMA, pltpu.SemaphoreType.DMA,
        pltpu.SemaphoreType.DMA, pltpu.SemaphoreType.DMA,
        pltpu.SemaphoreType.DMA, pltpu.SemaphoreType.DMA,
        pltpu.VMEM_SHARED((_NP, 128), jnp.float32),
    ],
)


def _gatden_body(as_hbm, ad_hbm, src_hbm, dst_hbm, m_hbm, zero_hbm,
                 out_hbm, ae_hbm,
                 sidx0, sidx1, didx0, didx1, asv0, asv1, adv0, adv1,
                 scv, mv,
                 ss0, ss1, sd0, sd1, ga0, ga1, gb0, gb1, acc):
    c = lax.axis_index("c")
    s = lax.axis_index("s")
    slab, nps = _prelude(zero_hbm, acc, s, 128)
    pltpu.sync_copy(m_hbm, mv)
    base0 = (c * 16 + s) * _EPT
    sidx = (sidx0, sidx1)
    didx = (didx0, didx1)
    asv = (asv0, asv1)
    adv = (adv0, adv1)
    ssem = (ss0, ss1)
    dsem = (sd0, sd1)
    asem = (ga0, ga1)
    bsem = (gb0, gb1)
    iota = lax.iota(jnp.int32, 16)
    mvec = mv[...]

    def zrow(e, carry):
        for cc in range(8):
            scv[e, pl.ds(cc * 16, 16)] = jnp.zeros((16,), jnp.float32)
        return carry

    lax.fori_loop(0, _ECH, zrow, 0)

    def idx_load(k, b):
        pltpu.async_copy(src_hbm.at[pl.ds(base0 + k * _ECH, _ECH)], sidx[b], ssem[b])
        pltpu.async_copy(dst_hbm.at[pl.ds(base0 + k * _ECH, _ECH)], didx[b], dsem[b])

    def idx_wait(b):
        pltpu.make_async_copy(src_hbm.at[pl.ds(0, _ECH)], sidx[b], ssem[b]).wait()
        pltpu.make_async_copy(dst_hbm.at[pl.ds(0, _ECH)], didx[b], dsem[b]).wait()

    def gath(b):
        pltpu.async_copy(as_hbm.at[sidx[b]], asv[b], asem[b])
        pltpu.async_copy(ad_hbm.at[didx[b]], adv[b], bsem[b])

    def gath_wait(b):
        pltpu.make_async_copy(as_hbm.at[sidx[b]], asv[b], asem[b]).wait()
        pltpu.make_async_copy(ad_hbm.at[didx[b]], adv[b], bsem[b]).wait()

    idx_load(0, 0)
    idx_wait(0)
    gath(0)
    idx_load(1, 1)

    def pair(kk, carry):
        for b in (0, 1):
            k = 2 * kk + b
            nb = 1 - b
            gath_wait(b)

            @pl.when(k + 1 < _NCH)
            def _():
                idx_wait(nb)
                gath(nb)

            for j in range(_ECH // 16):
                a = asv[b][pl.ds(j * 16, 16)] + adv[b][pl.ds(j * 16, 16)]
                a = jnp.where(a > 0.0, a, 0.2 * a)
                av = jnp.exp(jnp.minimum(a - mvec, 50.0))
                asv[b][pl.ds(j * 16, 16)] = av
                for ee in range(16):
                    scv[j * 16 + ee, pl.ds(0, 16)] = jnp.where(
                        iota == 0, av[ee], jnp.where(iota == 1, 1.0, 0.0))
            pltpu.sync_copy(asv[b], ae_hbm.at[pl.ds(base0 + k * _ECH, _ECH)])
            pltpu.sync_copy(scv, acc.at[didx[b]], add=True)

            @pl.when(k + 2 < _NCH)
            def _():
                idx_load(k + 2, b)
        return carry

    lax.fori_loop(0, _NCH // 2, pair, 0)
    _epilogue(acc, out_hbm, c, slab, nps)


_gatden = pl.kernel(
    _gatden_body,
    out_type=(jax.ShapeDtypeStruct((2, _NP, 128), jnp.float32),
              jax.ShapeDtypeStruct((_EPAD,), jnp.float32)),
    mesh=_MESH,
    scratch_types=[
        pltpu.VMEM((_ECH,), jnp.int32), pltpu.VMEM((_ECH,), jnp.int32),
        pltpu.VMEM((_ECH,), jnp.int32), pltpu.VMEM((_ECH,), jnp.int32),
        pltpu.VMEM((_ECH,), jnp.float32), pltpu.VMEM((_ECH,), jnp.float32),
        pltpu.VMEM((_ECH,), jnp.float32), pltpu.VMEM((_ECH,), jnp.float32),
        pltpu.VMEM((_ECH, 128), jnp.float32),
        pltpu.VMEM((16,), jnp.float32),
        pltpu.SemaphoreType.DMA, pltpu.SemaphoreType.DMA,
        pltpu.SemaphoreType.DMA, pltpu.SemaphoreType.DMA,
        pltpu.SemaphoreType.DMA, pltpu.SemaphoreType.DMA,
        pltpu.SemaphoreType.DMA, pltpu.SemaphoreType.DMA,
        pltpu.VMEM_SHARED((_NP, 128), jnp.float32),
    ],
)


def _gatnum_body(xw_hbm, den_hbm, ae_hbm, src_hbm, dst_hbm, zero_hbm, out_hbm,
                 sidx0, sidx1, didx0, didx1, aev0, aev1, dnv0, dnv1,
                 rows0, rows1,
                 ss0, ss1, sd0, sd1, ga0, ga1, gb0, gb1, gr0, gr1, acc):
    c = lax.axis_index("c")
    s = lax.axis_index("s")
    slab, nps = _prelude(zero_hbm, acc, s, 128)
    base0 = (c * 16 + s) * _EPT
    sidx = (sidx0, sidx1)
    didx = (didx0, didx1)
    aev = (aev0, aev1)
    dnv = (dnv0, dnv1)
    rows = (rows0, rows1)
    ssem = (ss0, ss1)
    dsem = (sd0, sd1)
    asem = (ga0, ga1)
    bsem = (gb0, gb1)
    rsem = (gr0, gr1)

    def idx_load(k, b):
        pltpu.async_copy(src_hbm.at[pl.ds(base0 + k * _ECH, _ECH)], sidx[b], ssem[b])
        pltpu.async_copy(dst_hbm.at[pl.ds(base0 + k * _ECH, _ECH)], didx[b], dsem[b])

    def idx_wait(b):
        pltpu.make_async_copy(src_hbm.at[pl.ds(0, _ECH)], sidx[b], ssem[b]).wait()
        pltpu.make_async_copy(dst_hbm.at[pl.ds(0, _ECH)], didx[b], dsem[b]).wait()

    def gath(k, b):
        pltpu.async_copy(ae_hbm.at[pl.ds(base0 + k * _ECH, _ECH)], aev[b], asem[b])
        pltpu.async_copy(den_hbm.at[didx[b]], dnv[b], bsem[b])
        pltpu.async_copy(xw_hbm.at[sidx[b]], rows[b], rsem[b])

    def gath_wait(b):
        pltpu.make_async_copy(ae_hbm.at[pl.ds(0, _ECH)], aev[b], asem[b]).wait()
        pltpu.make_async_copy(den_hbm.at[didx[b]], dnv[b], bsem[b]).wait()
        pltpu.make_async_copy(xw_hbm.at[sidx[b]], rows[b], rsem[b]).wait()

    idx_load(0, 0)
    idx_wait(0)
    gath(0, 0)
    idx_load(1, 1)

    def pair(kk, carry):
        for b in (0, 1):
            k = 2 * kk + b
            nb = 1 - b
            gath_wait(b)

            @pl.when(k + 1 < _NCH)
            def _():
                idx_wait(nb)
                gath(k + 1, nb)

            def grp(j, carry2):
                av = aev[b][pl.ds(j * 16, 16)] / dnv[b][pl.ds(j * 16, 16)]
                for ee in range(16):
                    e = j * 16 + ee
                    w = av[ee]
                    for cc in range(8):
                        rows[b][e, pl.ds(cc * 16, 16)] = (
                            rows[b][e, pl.ds(cc * 16, 16)] * w)
                return carry2

            lax.fori_loop(0, _ECH // 16, grp, 0)
            pltpu.sync_copy(rows[b], acc.at[didx[b]], add=True)

            @pl.when(k + 2 < _NCH)
            def _():
                idx_load(k + 2, b)
        return carry

    lax.fori_loop(0, _NCH // 2, pair, 0)
    _epilogue(acc, out_hbm, c, slab, nps)


_gatnum = pl.kernel(
    _gatnum_body,
    out_type=jax.ShapeDtypeStruct((2, _NP, 128), jnp.float32),
    mesh=_MESH,
    scratch_types=[
        pltpu.VMEM((_ECH,), jnp.int32), pltpu.VMEM((_ECH,), jnp.int32),
        pltpu.VMEM((_ECH,), jnp.int32), pltpu.VMEM((_ECH,), jnp.int32),
        pltpu.VMEM((_ECH,), jnp.float32), pltpu.VMEM((_ECH,), jnp.float32),
        pltpu.VMEM((_ECH,), jnp.float32), pltpu.VMEM((_ECH,), jnp.float32),
        pltpu.VMEM((_ECH, 128), jnp.float32), pltpu.VMEM((_ECH, 128), jnp.float32),
        pltpu.SemaphoreType.DMA, pltpu.SemaphoreType.DMA,
        pltpu.SemaphoreType.DMA, pltpu.SemaphoreType.DMA,
        pltpu.SemaphoreType.DMA, pltpu.SemaphoreType.DMA,
        pltpu.SemaphoreType.DMA, pltpu.SemaphoreType.DMA,
        pltpu.SemaphoreType.DMA, pltpu.SemaphoreType.DMA,
        pltpu.VMEM_SHARED((_NP, 128), jnp.float32),
    ],
)


def _erelu_body(u_hbm, v_hbm, src_hbm, dst_hbm, r_hbm,
                sidx0, sidx1, didx0, didx1, ru0, ru1, rv0, rv1,
                ss0, ss1, sd0, sd1, gu0, gu1, gv0, gv1):
    c = lax.axis_index("c")
    s = lax.axis_index("s")
    base0 = (c * 16 + s) * _EPT
    sidx = (sidx0, sidx1)
    didx = (didx0, didx1)
    ru = (ru0, ru1)
    rv = (rv0, rv1)
    ssem = (ss0, ss1)
    dsem = (sd0, sd1)
    usem = (gu0, gu1)
    vsem = (gv0, gv1)

    def idx_load(k, b):
        pltpu.async_copy(src_hbm.at[pl.ds(base0 + k * _ECH, _ECH)], sidx[b], ssem[b])
        pltpu.async_copy(dst_hbm.at[pl.ds(base0 + k * _ECH, _ECH)], didx[b], dsem[b])

    def idx_wait(b):
        pltpu.make_async_copy(src_hbm.at[pl.ds(0, _ECH)], sidx[b], ssem[b]).wait()
        pltpu.make_async_copy(dst_hbm.at[pl.ds(0, _ECH)], didx[b], dsem[b]).wait()

    def gath(b):
        pltpu.async_copy(u_hbm.at[sidx[b]], ru[b], usem[b])
        pltpu.async_copy(v_hbm.at[didx[b]], rv[b], vsem[b])

    def gath_wait(b):
        pltpu.make_async_copy(u_hbm.at[sidx[b]], ru[b], usem[b]).wait()
        pltpu.make_async_copy(v_hbm.at[didx[b]], rv[b], vsem[b]).wait()

    idx_load(0, 0)
    idx_wait(0)
    gath(0)
    idx_load(1, 1)

    def pair(kk, carry):
        for b in (0, 1):
            k = 2 * kk + b
            nb = 1 - b
            gath_wait(b)

            @pl.when(k + 1 < _NCH)
            def _():
                idx_wait(nb)
                gath(nb)

            def erow(e, carry2):
                for cc in range(8):
                    d = ru[b][e, pl.ds(cc * 16, 16)] - rv[b][e, pl.ds(cc * 16, 16)]
                    ru[b][e, pl.ds(cc * 16, 16)] = jnp.maximum(d, 0.0)
                return carry2

            lax.fori_loop(0, _ECH, erow, 0)
            pltpu.sync_copy(ru[b], r_hbm.at[pl.ds(base0 + k * _ECH, _ECH)])

            @pl.when(k + 2 < _NCH)
            def _():
                idx_load(k + 2, b)
        return carry

    lax.fori_loop(0, _NCH // 2, pair, 0)


_erelu = pl.kernel(
    _erelu_body,
    out_type=jax.ShapeDtypeStruct((_EPAD, 128), jnp.float32),
    mesh=_MESH,
    scratch_types=[
        pltpu.VMEM((_ECH,), jnp.int32), pltpu.VMEM((_ECH,), jnp.int32),
        pltpu.VMEM((_ECH,), jnp.int32), pltpu.VMEM((_ECH,), jnp.int32),
        pltpu.VMEM((_ECH, 128), jnp.float32), pltpu.VMEM((_ECH, 128), jnp.float32),
        pltpu.VMEM((_ECH, 128), jnp.float32), pltpu.VMEM((_ECH, 128), jnp.float32),
        pltpu.SemaphoreType.DMA, pltpu.SemaphoreType.DMA,
        pltpu.SemaphoreType.DMA, pltpu.SemaphoreType.DMA,
        pltpu.SemaphoreType.DMA, pltpu.SemaphoreType.DMA,
        pltpu.SemaphoreType.DMA, pltpu.SemaphoreType.DMA,
    ],
)


def kernel(x, pos, edge_index, ln1_w, ln1_b, ln2_w, ln2_b, gn1_w, gn1_b, gn2_w, gn2_b, gn3_w, gn3_b, gat_w, gat_asrc, gat_adst, gat_b, gcn1_w, gcn1_b, gcn2_w, gcn2_b, out_w, out_b):
    n, d = x.shape
    loops = jnp.arange(n, dtype=edge_index.dtype)
    ei = jnp.concatenate([edge_index, jnp.stack([loops, loops])], axis=1)
    src, dst = ei[0], ei[1]
    pad = jnp.full((_EPAD - src.shape[0],), _PADN, dtype=jnp.int32)
    srcp = jnp.concatenate([src.astype(jnp.int32), pad])
    dstp = jnp.concatenate([dst.astype(jnp.int32), pad])
    z128 = jnp.zeros((_NP, 128), jnp.float32)

    def npad(a):
        return jnp.pad(a, ((0, _NP - n),) + ((0, 0),) * (a.ndim - 1))

    u = x @ ln1_w[:d] + pos @ ln1_w[d:] + ln1_b
    v = pos @ ln1_w[d:]
    r = _erelu(npad(u), npad(v), srcp, dstp)
    h2 = r[: src.shape[0]] @ ln2_w
    agg = jax.ops.segment_max(h2, dst, num_segments=n) + ln2_b

    g = jax.nn.relu(agg @ gn1_w + gn1_b)
    g = jax.nn.relu(g @ gn2_w + gn2_b)
    h = g @ gn3_w + gn3_b

    xw = h @ gat_w
    a_s = (xw * gat_asrc).sum(-1)
    a_d = (xw * gat_adst).sum(-1)
    t = jnp.max(a_s) + jnp.max(a_d)
    m = jnp.where(t > 0, t, 0.2 * t)
    dh, ae = _gatden(npad(a_s), npad(a_d), srcp, dstp,
                     jnp.broadcast_to(m, (16,)), z128)
    dsum = dh[0] + dh[1]
    denom_full = dsum[:, 0]
    deg = dsum[:n, 1]
    nh = _gatnum(npad(xw), denom_full, ae, srcp, dstp, z128)
    num = (nh[0] + nh[1])[:n]
    h = jax.nn.relu(num + gat_b)

    dinv = jnp.where(deg > 0, deg ** -0.5, 0.0)

    def gcn(hh, W, b):
        p = npad(dinv[:, None] * hh)
        qh = _apass(p, srcp, dstp, z128)
        q = (qh[0] + qh[1])[:n]
        return (dinv[:, None] * q) @ W + b

    h = jax.nn.relu(gcn(h, gcn1_w, gcn1_b))
    h = jax.nn.relu(gcn(h, gcn2_w, gcn2_b))
    return gcn(h, out_w, out_b)
